# baseline (speedup 1.0000x reference)
import jax
import jax.numpy as jnp
from jax.experimental import pallas as pl

_K = 16
_DIMS = [34, 64, 64, 64, 64, 128, 128, 128, 128, 256, 256, 256, 256]
_B = 128
_P = 128
_N = _B * _P
_J = 8
_NJ = _B // _J
_EPS0 = 1e-5
_EPSL = 1e-5 * float(_K)


def _knn_kernel(pts_ref, as_ref):
    p = pts_ref[0]
    x = p[0:1, :]
    y = p[1:2, :]
    dx = jnp.transpose(x) - x
    dy = jnp.transpose(y) - y
    d2 = dx * dx + dy * dy
    r = jax.lax.broadcasted_iota(jnp.int32, (_P, _P), 0)
    c = jax.lax.broadcasted_iota(jnp.int32, (_P, _P), 1)
    d2 = jnp.where(r == c, 1e9, d2)
    adj = jnp.zeros((_P, _P), jnp.float32)
    for _ in range(_K):
        m = jnp.min(d2, axis=1, keepdims=True)
        sel = d2 <= m
        adj = adj + sel.astype(jnp.float32)
        d2 = jnp.where(sel, 1e9, d2)
    deg = jnp.sum(adj, axis=0, keepdims=True)
    c_src = jnp.where(deg > 0, jax.lax.rsqrt(deg), 1.0)
    as_ref[0] = adj * c_src


def _stats_kernel(x_ref, s_ref):
    i = pl.program_id(0)

    @pl.when(i == 0)
    def _():
        s_ref[...] = jnp.zeros_like(s_ref)

    x = x_ref[...]
    s_ref[0:1, :] += jnp.sum(x, axis=0, keepdims=True)
    s_ref[1:2, :] += jnp.sum(x * x, axis=0, keepdims=True)


def _make_layer_kernel(first, has_resid, emit_h, eps):
    def kfn(*refs):
        it = iter(refs)
        a_ref = next(it)
        st_ref = next(it)
        gb_ref = next(it)
        hp_ref = next(it) if has_resid else None
        as_ref = next(it)
        w_ref = next(it)
        agg_ref = next(it)
        sto_ref = next(it)
        ho_ref = next(it) if emit_h else None

        i = pl.program_id(0)
        a = a_ref[...]
        mean = st_ref[0:1, :] / _N
        var = st_ref[1:2, :] / _N - mean * mean
        scale = gb_ref[0:1, :] * jax.lax.rsqrt(var + eps)
        shift = gb_ref[1:2, :] - mean * scale
        h = a * scale + shift
        if not first:
            h = jnp.maximum(h, 0.0)
        if has_resid:
            h = h + hp_ref[...]
        if emit_h:
            ho_ref[...] = h
        y = jnp.dot(h, w_ref[...], preferred_element_type=jnp.float32)
        dout = y.shape[-1]
        y3 = y.reshape(_J, _P, dout)
        agg3 = jax.lax.dot_general(
            as_ref[...], y3, (((2,), (1,)), ((0,), (0,))),
            preferred_element_type=jnp.float32,
            precision=jax.lax.Precision.HIGHEST)
        agg = agg3.reshape(_J * _P, dout)
        agg_ref[...] = agg

        @pl.when(i == 0)
        def _():
            sto_ref[...] = jnp.zeros_like(sto_ref)

        sto_ref[0:1, :] += jnp.sum(agg, axis=0, keepdims=True)
        sto_ref[1:2, :] += jnp.sum(agg * agg, axis=0, keepdims=True)

    return kfn


def _final_kernel(a_ref, st_ref, gb_ref, hp_ref, mw0_ref, mb0_ref,
                  mw1_ref, mb1_ref, mw2_ref, mb2_ref, out_ref):
    a = a_ref[...]
    mean = st_ref[0:1, :] / _N
    var = st_ref[1:2, :] / _N - mean * mean
    scale = gb_ref[0:1, :] * jax.lax.rsqrt(var + _EPSL)
    shift = gb_ref[1:2, :] - mean * scale
    h = jnp.maximum(a * scale + shift, 0.0) + hp_ref[...]
    hg = jnp.mean(h.reshape(_J, _P, 256), axis=1)
    y = jnp.dot(hg, mw0_ref[...], preferred_element_type=jnp.float32)
    y = jnp.maximum(y + mb0_ref[0:1, :], 0.0)
    y = jnp.dot(y, mw1_ref[...], preferred_element_type=jnp.float32)
    y = jnp.maximum(y + mb1_ref[0:1, :], 0.0)
    y = jnp.dot(y, mw2_ref[...], preferred_element_type=jnp.float32)
    out_ref[...] = y + mb2_ref[0:1, :]


def _pack_gb(g, b):
    gb = jnp.stack([g, b], axis=0)
    return jnp.pad(gb, ((0, 6), (0, 0)))


def kernel(points, features, lorentz_vectors, mask, params):
    del lorentz_vectors, mask
    f32 = jnp.float32

    adj = pl.pallas_call(
        _knn_kernel,
        grid=(_B,),
        in_specs=[pl.BlockSpec((1, 2, _P), lambda i: (i, 0, 0))],
        out_specs=pl.BlockSpec((1, _P, _P), lambda i: (i, 0, 0)),
        out_shape=jax.ShapeDtypeStruct((_B, _P, _P), f32),
    )(points)

    h0 = jnp.transpose(features, (0, 2, 1)).reshape(_N, _DIMS[0])
    rows = _J * _P
    stats = pl.pallas_call(
        _stats_kernel,
        grid=(_NJ,),
        in_specs=[pl.BlockSpec((rows, _DIMS[0]), lambda i: (i, 0))],
        out_specs=pl.BlockSpec((8, _DIMS[0]), lambda i: (0, 0)),
        out_shape=jax.ShapeDtypeStruct((8, _DIMS[0]), f32),
    )(h0)

    a = h0
    gb = _pack_gb(params['bn_fts_gamma'], params['bn_fts_beta'])
    h_prev = None
    for i in range(12):
        din, dout = _DIMS[i], _DIMS[i + 1]
        first = (i == 0)
        has_resid = (not first) and (_DIMS[i - 1] == _DIMS[i])
        emit_h = (_DIMS[i] == _DIMS[i + 1])
        eps = _EPS0 if first else _EPSL

        in_specs = [
            pl.BlockSpec((rows, din), lambda i: (i, 0)),
            pl.BlockSpec((8, din), lambda i: (0, 0)),
            pl.BlockSpec((8, din), lambda i: (0, 0)),
        ]
        operands = [a, stats, gb]
        if has_resid:
            in_specs.append(pl.BlockSpec((rows, din), lambda i: (i, 0)))
            operands.append(h_prev)
        in_specs += [
            pl.BlockSpec((_J, _P, _P), lambda i: (i, 0, 0)),
            pl.BlockSpec((din, dout), lambda i: (0, 0)),
        ]
        operands += [adj, params[f'W{i}']]

        out_specs = [
            pl.BlockSpec((rows, dout), lambda i: (i, 0)),
            pl.BlockSpec((8, dout), lambda i: (0, 0)),
        ]
        out_shapes = [
            jax.ShapeDtypeStruct((_N, dout), f32),
            jax.ShapeDtypeStruct((8, dout), f32),
        ]
        if emit_h:
            out_specs.append(pl.BlockSpec((rows, din), lambda i: (i, 0)))
            out_shapes.append(jax.ShapeDtypeStruct((_N, din), f32))

        outs = pl.pallas_call(
            _make_layer_kernel(first, has_resid, emit_h, eps),
            grid=(_NJ,),
            in_specs=in_specs,
            out_specs=out_specs,
            out_shape=out_shapes,
        )(*operands)

        if emit_h:
            a, stats, h_prev = outs
        else:
            a, stats = outs
            h_prev = None
        gb = _pack_gb(params[f'g{i}'], params[f'be{i}'])

    mw2 = jnp.pad(params['MW2'], ((0, 0), (0, 128 - 5)))
    mb0 = params['Mb0'].reshape(1, 128)
    mb1 = params['Mb1'].reshape(1, 64)
    mb2 = jnp.pad(params['Mb2'], (0, 128 - 5)).reshape(1, 128)
    out = pl.pallas_call(
        _final_kernel,
        grid=(_NJ,),
        in_specs=[
            pl.BlockSpec((rows, 256), lambda i: (i, 0)),
            pl.BlockSpec((8, 256), lambda i: (0, 0)),
            pl.BlockSpec((8, 256), lambda i: (0, 0)),
            pl.BlockSpec((rows, 256), lambda i: (i, 0)),
            pl.BlockSpec((256, 128), lambda i: (0, 0)),
            pl.BlockSpec((1, 128), lambda i: (0, 0)),
            pl.BlockSpec((128, 64), lambda i: (0, 0)),
            pl.BlockSpec((1, 64), lambda i: (0, 0)),
            pl.BlockSpec((64, 128), lambda i: (0, 0)),
            pl.BlockSpec((1, 128), lambda i: (0, 0)),
        ],
        out_specs=pl.BlockSpec((_J, 128), lambda i: (i, 0)),
        out_shape=jax.ShapeDtypeStruct((_B, 128), f32),
    )(a, stats, gb, h_prev, params['MW0'], mb0, params['MW1'], mb1, mw2, mb2)
    return out[:, :5]


# speedup vs baseline: 1.3633x; 1.3633x over previous
import jax
import jax.numpy as jnp
from jax.experimental import pallas as pl

_K = 16
_DIMS = [34, 64, 64, 64, 64, 128, 128, 128, 128, 256, 256, 256, 256]
_B = 128
_P = 128
_N = _B * _P
_J = 8
_NJ = _B // _J
_EPS0 = 1e-5
_EPSL = 1e-5 * float(_K)


def _knn_kernel(pts_ref, as_ref):
    p = pts_ref[...]
    x = p[:, 0:1, :]
    y = p[:, 1:2, :]
    dx = jnp.transpose(x, (0, 2, 1)) - x
    dy = jnp.transpose(y, (0, 2, 1)) - y
    d2 = dx * dx + dy * dy
    r = jax.lax.broadcasted_iota(jnp.int32, (_J, _P, _P), 1)
    c = jax.lax.broadcasted_iota(jnp.int32, (_J, _P, _P), 2)
    d2 = jnp.where(r == c, 1e9, d2)
    m_t = jnp.zeros((_J, _P, _P), jnp.float32)
    for _ in range(_K):
        m = jnp.min(d2, axis=1, keepdims=True)
        sel = d2 <= m
        m_t = m_t + sel.astype(jnp.float32)
        d2 = jnp.where(sel, 1e9, d2)
    deg = jnp.sum(m_t, axis=2, keepdims=True)
    c_src = jnp.where(deg > 0, jax.lax.rsqrt(deg), 1.0)
    as_ref[...] = m_t * c_src


def _stats_kernel(x_ref, s_ref):
    i = pl.program_id(0)

    @pl.when(i == 0)
    def _():
        s_ref[...] = jnp.zeros_like(s_ref)

    x = x_ref[...]
    s_ref[0:1, :] += jnp.sum(x, axis=0, keepdims=True)
    s_ref[1:2, :] += jnp.sum(x * x, axis=0, keepdims=True)


def _make_layer_kernel(first, has_resid, emit_h, eps):
    def kfn(*refs):
        it = iter(refs)
        a_ref = next(it)
        st_ref = next(it)
        gb_ref = next(it)
        hp_ref = next(it) if has_resid else None
        as_ref = next(it)
        w_ref = next(it)
        agg_ref = next(it)
        sto_ref = next(it)
        ho_ref = next(it) if emit_h else None

        i = pl.program_id(0)
        a = a_ref[...]
        mean = st_ref[0:1, :] / _N
        var = st_ref[1:2, :] / _N - mean * mean
        scale = gb_ref[0:1, :] * jax.lax.rsqrt(var + eps)
        shift = gb_ref[1:2, :] - mean * scale
        h = a * scale + shift
        if not first:
            h = jnp.maximum(h, 0.0)
        if has_resid:
            h = h + hp_ref[...]
        if emit_h:
            ho_ref[...] = h
        y = jnp.dot(h, w_ref[...], preferred_element_type=jnp.float32)
        dout = y.shape[-1]
        y3 = y.reshape(_J, _P, dout)
        agg3 = jax.lax.dot_general(
            as_ref[...], y3, (((1,), (1,)), ((0,), (0,))),
            preferred_element_type=jnp.float32,
            precision=jax.lax.Precision.HIGHEST)
        agg = agg3.reshape(_J * _P, dout)
        agg_ref[...] = agg

        @pl.when(i == 0)
        def _():
            sto_ref[...] = jnp.zeros_like(sto_ref)

        sto_ref[0:1, :] += jnp.sum(agg, axis=0, keepdims=True)
        sto_ref[1:2, :] += jnp.sum(agg * agg, axis=0, keepdims=True)

    return kfn


def _final_kernel(a_ref, st_ref, gb_ref, hp_ref, mw0_ref, mb0_ref,
                  mw1_ref, mb1_ref, mw2_ref, mb2_ref, out_ref):
    a = a_ref[...]
    mean = st_ref[0:1, :] / _N
    var = st_ref[1:2, :] / _N - mean * mean
    scale = gb_ref[0:1, :] * jax.lax.rsqrt(var + _EPSL)
    shift = gb_ref[1:2, :] - mean * scale
    h = jnp.maximum(a * scale + shift, 0.0) + hp_ref[...]
    hg = jnp.mean(h.reshape(_J, _P, 256), axis=1)
    y = jnp.dot(hg, mw0_ref[...], preferred_element_type=jnp.float32)
    y = jnp.maximum(y + mb0_ref[0:1, :], 0.0)
    y = jnp.dot(y, mw1_ref[...], preferred_element_type=jnp.float32)
    y = jnp.maximum(y + mb1_ref[0:1, :], 0.0)
    y = jnp.dot(y, mw2_ref[...], preferred_element_type=jnp.float32)
    out_ref[...] = y + mb2_ref[0:1, :]


def _pack_gb(g, b):
    gb = jnp.stack([g, b], axis=0)
    return jnp.pad(gb, ((0, 6), (0, 0)))


def kernel(points, features, lorentz_vectors, mask, params):
    del lorentz_vectors, mask
    f32 = jnp.float32

    adj = pl.pallas_call(
        _knn_kernel,
        grid=(_NJ,),
        in_specs=[pl.BlockSpec((_J, 2, _P), lambda i: (i, 0, 0))],
        out_specs=pl.BlockSpec((_J, _P, _P), lambda i: (i, 0, 0)),
        out_shape=jax.ShapeDtypeStruct((_B, _P, _P), f32),
    )(points)

    h0 = jnp.transpose(features, (0, 2, 1)).reshape(_N, _DIMS[0])
    rows = _J * _P
    stats = pl.pallas_call(
        _stats_kernel,
        grid=(_NJ,),
        in_specs=[pl.BlockSpec((rows, _DIMS[0]), lambda i: (i, 0))],
        out_specs=pl.BlockSpec((8, _DIMS[0]), lambda i: (0, 0)),
        out_shape=jax.ShapeDtypeStruct((8, _DIMS[0]), f32),
    )(h0)

    a = h0
    gb = _pack_gb(params['bn_fts_gamma'], params['bn_fts_beta'])
    h_prev = None
    for i in range(12):
        din, dout = _DIMS[i], _DIMS[i + 1]
        first = (i == 0)
        has_resid = (not first) and (_DIMS[i - 1] == _DIMS[i])
        emit_h = (_DIMS[i] == _DIMS[i + 1])
        eps = _EPS0 if first else _EPSL

        in_specs = [
            pl.BlockSpec((rows, din), lambda i: (i, 0)),
            pl.BlockSpec((8, din), lambda i: (0, 0)),
            pl.BlockSpec((8, din), lambda i: (0, 0)),
        ]
        operands = [a, stats, gb]
        if has_resid:
            in_specs.append(pl.BlockSpec((rows, din), lambda i: (i, 0)))
            operands.append(h_prev)
        in_specs += [
            pl.BlockSpec((_J, _P, _P), lambda i: (i, 0, 0)),
            pl.BlockSpec((din, dout), lambda i: (0, 0)),
        ]
        operands += [adj, params[f'W{i}']]

        out_specs = [
            pl.BlockSpec((rows, dout), lambda i: (i, 0)),
            pl.BlockSpec((8, dout), lambda i: (0, 0)),
        ]
        out_shapes = [
            jax.ShapeDtypeStruct((_N, dout), f32),
            jax.ShapeDtypeStruct((8, dout), f32),
        ]
        if emit_h:
            out_specs.append(pl.BlockSpec((rows, din), lambda i: (i, 0)))
            out_shapes.append(jax.ShapeDtypeStruct((_N, din), f32))

        outs = pl.pallas_call(
            _make_layer_kernel(first, has_resid, emit_h, eps),
            grid=(_NJ,),
            in_specs=in_specs,
            out_specs=out_specs,
            out_shape=out_shapes,
        )(*operands)

        if emit_h:
            a, stats, h_prev = outs
        else:
            a, stats = outs
            h_prev = None
        gb = _pack_gb(params[f'g{i}'], params[f'be{i}'])

    mw2 = jnp.pad(params['MW2'], ((0, 0), (0, 128 - 5)))
    mb0 = params['Mb0'].reshape(1, 128)
    mb1 = params['Mb1'].reshape(1, 64)
    mb2 = jnp.pad(params['Mb2'], (0, 128 - 5)).reshape(1, 128)
    out = pl.pallas_call(
        _final_kernel,
        grid=(_NJ,),
        in_specs=[
            pl.BlockSpec((rows, 256), lambda i: (i, 0)),
            pl.BlockSpec((8, 256), lambda i: (0, 0)),
            pl.BlockSpec((8, 256), lambda i: (0, 0)),
            pl.BlockSpec((rows, 256), lambda i: (i, 0)),
            pl.BlockSpec((256, 128), lambda i: (0, 0)),
            pl.BlockSpec((1, 128), lambda i: (0, 0)),
            pl.BlockSpec((128, 64), lambda i: (0, 0)),
            pl.BlockSpec((1, 64), lambda i: (0, 0)),
            pl.BlockSpec((64, 128), lambda i: (0, 0)),
            pl.BlockSpec((1, 128), lambda i: (0, 0)),
        ],
        out_specs=pl.BlockSpec((_J, 128), lambda i: (i, 0)),
        out_shape=jax.ShapeDtypeStruct((_B, 128), f32),
    )(a, stats, gb, h_prev, params['MW0'], mb0, params['MW1'], mb1, mw2, mb2)
    return out[:, :5]
